# Initial kernel scaffold; baseline (speedup 1.0000x reference)
import jax
import jax.numpy as jnp
from jax.experimental import pallas as pl


def kernel(edges, node_features, edge_features, edge_encoder_params, node_encoder_params, edge_processor_params, node_processor_params, decoder_params):
    raise NotImplementedError("write your pallas kernel here")


# speedup vs baseline: 10.3897x; 10.3897x over previous
import functools

import jax
import jax.numpy as jnp
from jax import lax
from jax.experimental import pallas as pl
from jax.experimental.pallas import tpu as pltpu
from jax.experimental.pallas import tpu_sc as plsc

_NC = 2
_NS = 16
_CH = 80
_ZR = 80


def _node_tables_body(x_ref, wn1, bn1, wn2, bn2, wa, ba, wb, bb,
                      enc_ref, a_ref, b_ref):
    x = x_ref[...]
    t = jnp.maximum(jnp.dot(x, wn1[...], preferred_element_type=jnp.float32)
                    + bn1[...], 0.0)
    enc_ref[...] = jnp.dot(t, wn2[...], preferred_element_type=jnp.float32) + bn2[...]
    a_ref[...] = jnp.dot(t, wa[...], preferred_element_type=jnp.float32) + ba[...]
    b_ref[...] = jnp.dot(t, wb[...], preferred_element_type=jnp.float32) + bb[...]


def _edge_q_body(ef_ref, we1, be1, wf, bf, q_ref):
    t = jnp.maximum(jnp.dot(ef_ref[...], we1[...],
                            preferred_element_type=jnp.float32) + be1[...], 0.0)
    q_ref[...] = jnp.dot(t, wf[...], preferred_element_type=jnp.float32) + bf[...]


def _node_out_body(nf_ref, enc_ref, s_ref, cnt_ref, v1a, w2v1b, b2v1b, c1,
                   v2d1, cc, d2, d2b, out_ref):
    cnt_col = cnt_ref[...][:, 0:1]
    hmid = jnp.maximum(
        jnp.dot(enc_ref[...], v1a[...], preferred_element_type=jnp.float32)
        + jnp.dot(s_ref[...], w2v1b[...], preferred_element_type=jnp.float32)
        + cnt_col * b2v1b[...]
        + c1[...], 0.0)
    g = jnp.maximum(
        jnp.dot(hmid, v2d1[...], preferred_element_type=jnp.float32) + cc[...], 0.0)
    out_ref[...] = (jnp.dot(g, d2[...], preferred_element_type=jnp.float32)
                    + d2b[...] + nf_ref[...])


_SUP = 10


def _sc_segment_sum(edges_dst, edges_src, q, a, b, n_nodes):
    batch, _, n_sup, _, _ = edges_dst.shape
    hdim = q.shape[-1]
    zgroups = n_nodes // _ZR
    edges_per_tile = n_sup * _SUP * _CH

    def body(dst_hbm, src_hbm, q_hbm, a_hbm, b_hbm, s_out, cnt_out,
             S, idxd, idxs, bq, ba, bb, sem_a, sem_b, sem_q):
        c = lax.axis_index("c")
        s = lax.axis_index("s")
        zvec = jnp.zeros((16,), jnp.float32)

        def zero_ba(i, _):
            for j in range(hdim // 16):
                ba[i, pl.ds(j * 16, 16)] = zvec
            return 0

        lax.fori_loop(0, _ZR, zero_ba, 0)

        def zero_group(m, _):
            g = s + m * _NS

            @pl.when(g < zgroups)
            def _():
                pltpu.sync_copy(ba, S.at[pl.ds(g * _ZR, _ZR)])

            return 0

        zg_iters = (zgroups + _NS - 1) // _NS
        lax.fori_loop(0, zg_iters, zero_group, 0)
        plsc.subcore_barrier()

        qbase = (c * _NS + s) * edges_per_tile

        def superchunk(m, _):
            pltpu.sync_copy(dst_hbm.at[c, s, m], idxd)
            pltpu.sync_copy(src_hbm.at[c, s, m], idxs)

            def chunk(kk, _):
                off = qbase + (m * _SUP + kk) * _CH
                cpa = pltpu.async_copy(a_hbm.at[c].at[idxd.at[kk]], ba, sem_a)
                cpb = pltpu.async_copy(b_hbm.at[c].at[idxs.at[kk]], bb, sem_b)
                cpq = pltpu.async_copy(q_hbm.at[pl.ds(off, _CH)], bq, sem_q)
                cpa.wait()
                cpb.wait()
                cpq.wait()

                def row(i, _):
                    for j in range(hdim // 16):
                        sl = pl.ds(j * 16, 16)
                        bq[i, sl] = jnp.maximum(
                            bq[i, sl] + ba[i, sl] + bb[i, sl], 0.0)
                    return 0

                lax.fori_loop(0, _CH, row, 0)
                pltpu.sync_copy(bq, S.at[idxd.at[kk]], add=True)
                return 0

            lax.fori_loop(0, _SUP, chunk, 0)
            return 0

        lax.fori_loop(0, n_sup, superchunk, 0)
        plsc.subcore_barrier()

        def zero_ba_again(i, _):
            for j in range(hdim // 16):
                ba[i, pl.ds(j * 16, 16)] = zvec
            return 0

        lax.fori_loop(0, _ZR, zero_ba_again, 0)

        def out_group(m, _):
            g = s + m * _NS

            @pl.when(g < zgroups)
            def _():
                pltpu.sync_copy(S.at[pl.ds(g * _ZR, _ZR)], bb)
                pltpu.sync_copy(bb, s_out.at[c, pl.ds(g * _ZR, _ZR)])
                pltpu.sync_copy(ba, S.at[pl.ds(g * _ZR, _ZR)])

            return 0

        lax.fori_loop(0, zg_iters, out_group, 0)
        plsc.subcore_barrier()

        onev = jnp.where(lax.iota(jnp.int32, 16) == 0, 1.0, 0.0).astype(
            jnp.float32)

        def onehot_row(i, _):
            bq[i, pl.ds(0, 16)] = onev
            for j in range(1, hdim // 16):
                bq[i, pl.ds(j * 16, 16)] = zvec
            return 0

        lax.fori_loop(0, _CH, onehot_row, 0)

        def superchunk2(m, _):
            pltpu.sync_copy(dst_hbm.at[c, s, m], idxd)

            def chunk(kk, _):
                pltpu.sync_copy(bq, S.at[idxd.at[kk]], add=True)
                return 0

            lax.fori_loop(0, _SUP, chunk, 0)
            return 0

        lax.fori_loop(0, n_sup, superchunk2, 0)
        plsc.subcore_barrier()

        def cnt_group(m, _):
            g = s + m * _NS

            @pl.when(g < zgroups)
            def _():
                pltpu.sync_copy(S.at[pl.ds(g * _ZR, _ZR)], bb)
                pltpu.sync_copy(bb, cnt_out.at[c, pl.ds(g * _ZR, _ZR)])

            return 0

        lax.fori_loop(0, zg_iters, cnt_group, 0)

    mesh = plsc.VectorSubcoreMesh(core_axis_name="c", subcore_axis_name="s")
    fn = pl.kernel(
        body,
        out_type=(jax.ShapeDtypeStruct((batch, n_nodes, hdim), jnp.float32),
                  jax.ShapeDtypeStruct((batch, n_nodes, hdim), jnp.float32)),
        mesh=mesh,
        scratch_types=[
            pltpu.VMEM_SHARED((n_nodes, hdim), jnp.float32),
            pltpu.VMEM((_SUP, _CH), jnp.int32),
            pltpu.VMEM((_SUP, _CH), jnp.int32),
            pltpu.VMEM((_CH, hdim), jnp.float32),
            pltpu.VMEM((_CH, hdim), jnp.float32),
            pltpu.VMEM((_CH, hdim), jnp.float32),
            pltpu.SemaphoreType.DMA,
            pltpu.SemaphoreType.DMA,
            pltpu.SemaphoreType.DMA,
        ],
        compiler_params=pltpu.CompilerParams(needs_layout_passes=False),
        name="gnn_segment_relu_scatter",
    )
    return fn(edges_dst, edges_src, q, a, b)


def _tc_call(body, grid, in_specs, out_specs, out_shapes, name):
    return pl.pallas_call(
        body,
        grid=grid,
        in_specs=in_specs,
        out_specs=out_specs,
        out_shape=out_shapes,
        name=name,
    )


def kernel(edges, node_features, edge_features, edge_encoder_params,
           node_encoder_params, edge_processor_params, node_processor_params,
           decoder_params):
    we1, be1, we2, be2 = edge_encoder_params
    wn1, bn1, wn2, bn2 = node_encoder_params
    wp1, bp1, wp2, bp2 = edge_processor_params
    wq1, bq1, wq2, bq2 = node_processor_params
    wd1, bd1, wd2, bd2 = decoder_params

    node_features = node_features.astype(jnp.float32)
    edge_features = edge_features.astype(jnp.float32)

    batch, n_nodes, d_node = node_features.shape
    _, n_edges, d_edge = edge_features.shape
    hdim = wn2.shape[1]

    w1a, w1b, w1c = wp1[:hdim], wp1[hdim:2 * hdim], wp1[2 * hdim:]
    wf = we2 @ w1a
    bf = be2 @ w1a + bp1
    wa = wn2 @ w1b
    ba_ = bn2 @ w1b
    wb = wn2 @ w1c
    bb_ = bn2 @ w1c
    v1a, v1b = wq1[:hdim], wq1[hdim:]
    w2v1b = wp2 @ v1b
    b2v1b = (bp2 @ v1b).reshape(1, hdim)
    v2d1 = wq2 @ wd1
    cc = bq2 @ wd1 + bd1

    def row2d(v):
        return v.reshape(1, -1)

    nm = batch * n_nodes
    blk_n = 2000
    assert nm % blk_n == 0
    nf_flat = node_features.reshape(nm, d_node)
    dspec = pl.BlockSpec((blk_n, d_node), lambda i: (i, 0))
    ospec = pl.BlockSpec((blk_n, hdim), lambda i: (i, 0))

    def wspec(shape):
        return pl.BlockSpec(shape, lambda i: tuple(0 for _ in shape))

    enc, atab, btab = _tc_call(
        _node_tables_body, (nm // blk_n,),
        [dspec,
         wspec(wn1.shape), wspec((1, hdim)), wspec(wn2.shape), wspec((1, hdim)),
         wspec(wa.shape), wspec((1, hdim)), wspec(wb.shape), wspec((1, hdim))],
        [ospec, ospec, ospec],
        [jax.ShapeDtypeStruct((nm, hdim), jnp.float32)] * 3,
        "gnn_node_tables",
    )(nf_flat, wn1, row2d(bn1), wn2, row2d(bn2), wa, row2d(ba_), wb, row2d(bb_))

    em = batch * n_edges
    blk_e = 2000
    assert em % blk_e == 0
    ef_flat = edge_features.reshape(em, d_edge)
    q = _tc_call(
        _edge_q_body, (em // blk_e,),
        [pl.BlockSpec((blk_e, d_edge), lambda i: (i, 0)),
         wspec(we1.shape), wspec((1, hdim)), wspec(wf.shape), wspec((1, hdim))],
        pl.BlockSpec((blk_e, hdim), lambda i: (i, 0)),
        jax.ShapeDtypeStruct((em, hdim), jnp.float32),
        "gnn_edge_q",
    )(ef_flat, we1, row2d(be1), wf, row2d(bf))

    ept = n_edges // _NS
    n_sup = ept // (_SUP * _CH)
    assert ept % (_SUP * _CH) == 0
    dst = edges[:, 0, :].reshape(batch, _NS, n_sup, _SUP, _CH)
    src = edges[:, 1, :].reshape(batch, _NS, n_sup, _SUP, _CH)
    s_acc, cnt = _sc_segment_sum(dst, src, q,
                                 atab.reshape(batch, n_nodes, hdim),
                                 btab.reshape(batch, n_nodes, hdim),
                                 n_nodes)

    s_flat = s_acc.reshape(nm, hdim)
    cnt_flat = cnt.reshape(nm, hdim)
    out = _tc_call(
        _node_out_body, (nm // blk_n,),
        [dspec, ospec, ospec, ospec,
         wspec(v1a.shape), wspec(w2v1b.shape), wspec((1, hdim)),
         wspec((1, hdim)), wspec(v2d1.shape), wspec((1, hdim)),
         wspec(wd2.shape), wspec((1, d_node))],
        pl.BlockSpec((blk_n, d_node), lambda i: (i, 0)),
        jax.ShapeDtypeStruct((nm, d_node), jnp.float32),
        "gnn_node_out",
    )(nf_flat, enc, s_flat, cnt_flat, v1a, w2v1b, b2v1b, row2d(bq1), v2d1,
      row2d(cc), wd2, row2d(bd2))

    return out.reshape(batch, n_nodes, d_node)


# speedup vs baseline: 11.6231x; 1.1187x over previous
import functools

import jax
import jax.numpy as jnp
from jax import lax
from jax.experimental import pallas as pl
from jax.experimental.pallas import tpu as pltpu
from jax.experimental.pallas import tpu_sc as plsc

_NC = 2
_NS = 16
_CH = 40
_ZR = 40


def _node_tables_body(x_ref, wn1, bn1, wn2, bn2, wa, ba, wb, bb,
                      enc_ref, a_ref, b_ref):
    x = x_ref[...]
    t = jnp.maximum(jnp.dot(x, wn1[...], preferred_element_type=jnp.float32)
                    + bn1[...], 0.0)
    enc_ref[...] = jnp.dot(t, wn2[...], preferred_element_type=jnp.float32) + bn2[...]
    a_ref[...] = jnp.dot(t, wa[...], preferred_element_type=jnp.float32) + ba[...]
    b_ref[...] = jnp.dot(t, wb[...], preferred_element_type=jnp.float32) + bb[...]


def _edge_q_body(ef_ref, we1, be1, wf, bf, q_ref):
    t = jnp.maximum(jnp.dot(ef_ref[...], we1[...],
                            preferred_element_type=jnp.float32) + be1[...], 0.0)
    q_ref[...] = jnp.dot(t, wf[...], preferred_element_type=jnp.float32) + bf[...]


def _node_out_body(nf_ref, enc_ref, s_ref, cnt_ref, v1a, w2v1b, b2v1b, c1,
                   v2d1, cc, d2, d2b, out_ref):
    cnt_col = cnt_ref[...][:, 0:1]
    hmid = jnp.maximum(
        jnp.dot(enc_ref[...], v1a[...], preferred_element_type=jnp.float32)
        + jnp.dot(s_ref[...], w2v1b[...], preferred_element_type=jnp.float32)
        + cnt_col * b2v1b[...]
        + c1[...], 0.0)
    g = jnp.maximum(
        jnp.dot(hmid, v2d1[...], preferred_element_type=jnp.float32) + cc[...], 0.0)
    out_ref[...] = (jnp.dot(g, d2[...], preferred_element_type=jnp.float32)
                    + d2b[...] + nf_ref[...])


_SUP = 20


def _sc_segment_sum(edges_dst, edges_src, q, a, b, n_nodes):
    batch, _, n_sup, _, _ = edges_dst.shape
    hdim = q.shape[-1]
    zgroups = n_nodes // _ZR
    edges_per_tile = n_sup * _SUP * _CH

    def body(dst_hbm, src_hbm, q_hbm, a_hbm, b_hbm, s_out, cnt_out,
             S, idxd, idxs, bq0, bq1, ba0, ba1, bb0, bb1,
             sem_g0, sem_g1, sem_s0, sem_s1):
        c = lax.axis_index("c")
        s = lax.axis_index("s")
        zvec = jnp.zeros((16,), jnp.float32)
        bqs, bas, bbs = (bq0, bq1), (ba0, ba1), (bb0, bb1)
        sgs, sss = (sem_g0, sem_g1), (sem_s0, sem_s1)

        def fill(buf, vec):
            def frow(i, _):
                for j in range(hdim // 16):
                    buf[i, pl.ds(j * 16, 16)] = vec
                return 0

            lax.fori_loop(0, _CH, frow, 0)

        fill(ba0, zvec)

        def zero_group(m, _):
            g = s + m * _NS

            @pl.when(g < zgroups)
            def _():
                pltpu.sync_copy(ba0, S.at[pl.ds(g * _ZR, _ZR)])

            return 0

        zg_iters = (zgroups + _NS - 1) // _NS
        lax.fori_loop(0, zg_iters, zero_group, 0)
        plsc.subcore_barrier()

        qbase = (c * _NS + s) * edges_per_tile

        def gather_issue(m, kk, slot):
            off = qbase + (m * _SUP + kk) * _CH
            pltpu.async_copy(a_hbm.at[c].at[idxd.at[kk]], bas[slot], sgs[slot])
            pltpu.async_copy(b_hbm.at[c].at[idxs.at[kk]], bbs[slot], sgs[slot])
            pltpu.async_copy(q_hbm.at[pl.ds(off, _CH)], bqs[slot], sgs[slot])

        def gather_wait(m, kk, slot):
            off = qbase + (m * _SUP + kk) * _CH
            pltpu.make_async_copy(
                a_hbm.at[c].at[idxd.at[kk]], bas[slot], sgs[slot]).wait()
            pltpu.make_async_copy(
                b_hbm.at[c].at[idxs.at[kk]], bbs[slot], sgs[slot]).wait()
            pltpu.make_async_copy(
                q_hbm.at[pl.ds(off, _CH)], bqs[slot], sgs[slot]).wait()

        def scatter_wait(kk, slot):
            pltpu.make_async_copy(bqs[slot], S.at[idxd.at[kk]],
                                  sss[slot]).wait()

        def superchunk(m, _):
            @pl.when(m >= 1)
            def _():
                scatter_wait(_SUP - 2, 0)
                scatter_wait(_SUP - 1, 1)

            pltpu.sync_copy(dst_hbm.at[c, s, m], idxd)
            pltpu.sync_copy(src_hbm.at[c, s, m], idxs)
            gather_issue(m, 0, 0)

            def pair(mm, _):
                for bslot in (0, 1):
                    kk = mm * 2 + bslot
                    gather_wait(m, kk, bslot)
                    if bslot == 0:
                        gather_issue(m, kk + 1, 1)
                    else:
                        @pl.when(mm < _SUP // 2 - 1)
                        def _():
                            gather_issue(m, kk + 1, 0)

                    @pl.when(mm >= 1)
                    def _():
                        scatter_wait(kk - 2, bslot)

                    bq_, ba_, bb_ = bqs[bslot], bas[bslot], bbs[bslot]

                    def row(i, _):
                        for j in range(hdim // 16):
                            sl = pl.ds(j * 16, 16)
                            bq_[i, sl] = jnp.maximum(
                                bq_[i, sl] + ba_[i, sl] + bb_[i, sl], 0.0)
                        return 0

                    lax.fori_loop(0, _CH, row, 0)
                    pltpu.async_copy(bq_, S.at[idxd.at[kk]], sss[bslot],
                                     add=True)
                return 0

            lax.fori_loop(0, _SUP // 2, pair, 0)
            return 0

        lax.fori_loop(0, n_sup, superchunk, 0)
        scatter_wait(_SUP - 2, 0)
        scatter_wait(_SUP - 1, 1)
        plsc.subcore_barrier()

        fill(ba0, zvec)

        def out_group(m, _):
            g = s + m * _NS

            @pl.when(g < zgroups)
            def _():
                pltpu.sync_copy(S.at[pl.ds(g * _ZR, _ZR)], bb0)
                pltpu.sync_copy(bb0, s_out.at[c, pl.ds(g * _ZR, _ZR)])
                pltpu.sync_copy(ba0, S.at[pl.ds(g * _ZR, _ZR)])

            return 0

        lax.fori_loop(0, zg_iters, out_group, 0)
        plsc.subcore_barrier()

        onev = jnp.where(lax.iota(jnp.int32, 16) == 0, 1.0, 0.0).astype(
            jnp.float32)
        fill(bq0, zvec)

        def one_row(i, _):
            bq0[i, pl.ds(0, 16)] = onev
            return 0

        lax.fori_loop(0, _CH, one_row, 0)

        def superchunk2(m, _):
            @pl.when(m >= 1)
            def _():
                for kk in range(_SUP):
                    pltpu.make_async_copy(bq0, S.at[idxd.at[kk]],
                                          sem_s0).wait()

            pltpu.sync_copy(dst_hbm.at[c, s, m], idxd)
            for kk in range(_SUP):
                pltpu.async_copy(bq0, S.at[idxd.at[kk]], sem_s0, add=True)
            return 0

        lax.fori_loop(0, n_sup, superchunk2, 0)
        for kk in range(_SUP):
            pltpu.make_async_copy(bq0, S.at[idxd.at[kk]], sem_s0).wait()
        plsc.subcore_barrier()

        def cnt_group(m, _):
            g = s + m * _NS

            @pl.when(g < zgroups)
            def _():
                pltpu.sync_copy(S.at[pl.ds(g * _ZR, _ZR)], bb0)
                pltpu.sync_copy(bb0, cnt_out.at[c, pl.ds(g * _ZR, _ZR)])

            return 0

        lax.fori_loop(0, zg_iters, cnt_group, 0)

    mesh = plsc.VectorSubcoreMesh(core_axis_name="c", subcore_axis_name="s")
    fn = pl.kernel(
        body,
        out_type=(jax.ShapeDtypeStruct((batch, n_nodes, hdim), jnp.float32),
                  jax.ShapeDtypeStruct((batch, n_nodes, hdim), jnp.float32)),
        mesh=mesh,
        scratch_types=[
            pltpu.VMEM_SHARED((n_nodes, hdim), jnp.float32),
            pltpu.VMEM((_SUP, _CH), jnp.int32),
            pltpu.VMEM((_SUP, _CH), jnp.int32),
            pltpu.VMEM((_CH, hdim), jnp.float32),
            pltpu.VMEM((_CH, hdim), jnp.float32),
            pltpu.VMEM((_CH, hdim), jnp.float32),
            pltpu.VMEM((_CH, hdim), jnp.float32),
            pltpu.VMEM((_CH, hdim), jnp.float32),
            pltpu.VMEM((_CH, hdim), jnp.float32),
            pltpu.SemaphoreType.DMA,
            pltpu.SemaphoreType.DMA,
            pltpu.SemaphoreType.DMA,
            pltpu.SemaphoreType.DMA,
        ],
        compiler_params=pltpu.CompilerParams(needs_layout_passes=False),
        name="gnn_segment_relu_scatter",
    )
    return fn(edges_dst, edges_src, q, a, b)


def _tc_call(body, grid, in_specs, out_specs, out_shapes, name):
    return pl.pallas_call(
        body,
        grid=grid,
        in_specs=in_specs,
        out_specs=out_specs,
        out_shape=out_shapes,
        name=name,
    )


def kernel(edges, node_features, edge_features, edge_encoder_params,
           node_encoder_params, edge_processor_params, node_processor_params,
           decoder_params):
    we1, be1, we2, be2 = edge_encoder_params
    wn1, bn1, wn2, bn2 = node_encoder_params
    wp1, bp1, wp2, bp2 = edge_processor_params
    wq1, bq1, wq2, bq2 = node_processor_params
    wd1, bd1, wd2, bd2 = decoder_params

    node_features = node_features.astype(jnp.float32)
    edge_features = edge_features.astype(jnp.float32)

    batch, n_nodes, d_node = node_features.shape
    _, n_edges, d_edge = edge_features.shape
    hdim = wn2.shape[1]

    w1a, w1b, w1c = wp1[:hdim], wp1[hdim:2 * hdim], wp1[2 * hdim:]
    wf = we2 @ w1a
    bf = be2 @ w1a + bp1
    wa = wn2 @ w1b
    ba_ = bn2 @ w1b
    wb = wn2 @ w1c
    bb_ = bn2 @ w1c
    v1a, v1b = wq1[:hdim], wq1[hdim:]
    w2v1b = wp2 @ v1b
    b2v1b = (bp2 @ v1b).reshape(1, hdim)
    v2d1 = wq2 @ wd1
    cc = bq2 @ wd1 + bd1

    def row2d(v):
        return v.reshape(1, -1)

    nm = batch * n_nodes
    blk_n = 2000
    assert nm % blk_n == 0
    nf_flat = node_features.reshape(nm, d_node)
    dspec = pl.BlockSpec((blk_n, d_node), lambda i: (i, 0))
    ospec = pl.BlockSpec((blk_n, hdim), lambda i: (i, 0))

    def wspec(shape):
        return pl.BlockSpec(shape, lambda i: tuple(0 for _ in shape))

    enc, atab, btab = _tc_call(
        _node_tables_body, (nm // blk_n,),
        [dspec,
         wspec(wn1.shape), wspec((1, hdim)), wspec(wn2.shape), wspec((1, hdim)),
         wspec(wa.shape), wspec((1, hdim)), wspec(wb.shape), wspec((1, hdim))],
        [ospec, ospec, ospec],
        [jax.ShapeDtypeStruct((nm, hdim), jnp.float32)] * 3,
        "gnn_node_tables",
    )(nf_flat, wn1, row2d(bn1), wn2, row2d(bn2), wa, row2d(ba_), wb, row2d(bb_))

    em = batch * n_edges
    blk_e = 2000
    assert em % blk_e == 0
    ef_flat = edge_features.reshape(em, d_edge)
    q = _tc_call(
        _edge_q_body, (em // blk_e,),
        [pl.BlockSpec((blk_e, d_edge), lambda i: (i, 0)),
         wspec(we1.shape), wspec((1, hdim)), wspec(wf.shape), wspec((1, hdim))],
        pl.BlockSpec((blk_e, hdim), lambda i: (i, 0)),
        jax.ShapeDtypeStruct((em, hdim), jnp.float32),
        "gnn_edge_q",
    )(ef_flat, we1, row2d(be1), wf, row2d(bf))

    ept = n_edges // _NS
    n_sup = ept // (_SUP * _CH)
    assert ept % (_SUP * _CH) == 0
    dst = edges[:, 0, :].reshape(batch, _NS, n_sup, _SUP, _CH)
    src = edges[:, 1, :].reshape(batch, _NS, n_sup, _SUP, _CH)
    s_acc, cnt = _sc_segment_sum(dst, src, q,
                                 atab.reshape(batch, n_nodes, hdim),
                                 btab.reshape(batch, n_nodes, hdim),
                                 n_nodes)

    s_flat = s_acc.reshape(nm, hdim)
    cnt_flat = cnt.reshape(nm, hdim)
    out = _tc_call(
        _node_out_body, (nm // blk_n,),
        [dspec, ospec, ospec, ospec,
         wspec(v1a.shape), wspec(w2v1b.shape), wspec((1, hdim)),
         wspec((1, hdim)), wspec(v2d1.shape), wspec((1, hdim)),
         wspec(wd2.shape), wspec((1, d_node))],
        pl.BlockSpec((blk_n, d_node), lambda i: (i, 0)),
        jax.ShapeDtypeStruct((nm, d_node), jnp.float32),
        "gnn_node_out",
    )(nf_flat, enc, s_flat, cnt_flat, v1a, w2v1b, b2v1b, row2d(bq1), v2d1,
      row2d(cc), wd2, row2d(bd2))

    return out.reshape(batch, n_nodes, d_node)
